# baseline (speedup 1.0000x reference)
import functools

import jax
import jax.numpy as jnp
from jax import lax
from jax.experimental import pallas as pl
from jax.experimental.pallas import tpu as pltpu
from jax.experimental.pallas import tpu_sc as plsc

B = 4
Z = 96
YX = 96
NW = 32
WPB = NW // B
ZPW = Z // WPB
LANES = 16
VPR = YX // LANES
NLAB = 64
BINS = NLAB * LANES
TEMP = 0.2


def _sc_segsum_body(pred_hbm, lab_hbm, sum_out, cnt_out,
                    lab0, lab1, a0, a1, b0, b1, accs, accc, sem0, sem1):
    cid = lax.axis_index("c")
    sid = lax.axis_index("s")
    wid = cid * 16 + sid
    b = wid // WPB
    z0 = (wid % WPB) * ZPW

    zeros = jnp.zeros((LANES,), jnp.float32)

    @plsc.parallel_loop(0, NLAB, unroll=8)
    def _(j):
        accs[pl.ds(j * LANES, LANES)] = zeros
        accc[pl.ds(j * LANES, LANES)] = zeros

    labs = [lab0, lab1]
    avs = [a0, a1]
    bvs = [b0, b1]
    sems = [sem0, sem1]

    def start(k, bank):
        z = z0 + k
        return (
            pltpu.async_copy(lab_hbm.at[b, 0, z], labs[bank], sems[bank]),
            pltpu.async_copy(pred_hbm.at[b, 0, z], avs[bank], sems[bank]),
            pltpu.async_copy(pred_hbm.at[b, 1, z], bvs[bank], sems[bank]),
        )

    iota = lax.iota(jnp.int32, LANES)
    ones = jnp.ones((LANES,), jnp.float32)

    def compute(bank):
        labr, ar, br = labs[bank], avs[bank], bvs[bank]

        @plsc.parallel_loop(0, YX, unroll=8)
        def _(r):
            for c in range(VPR):
                sl = pl.ds(c * LANES, LANES)
                li = labr[r, sl].astype(jnp.int32)
                d = ar[r, sl] - br[r, sl]
                fg = 1.0 / (1.0 + jnp.exp(d))
                idx = li * LANES + iota
                plsc.addupdate_scatter(accs, [idx], fg)
                plsc.addupdate_scatter(accc, [idx], ones)

    handles = [None, None]
    handles[0] = start(0, 0)
    for k in range(ZPW):
        bank = k % 2
        if k + 1 < ZPW:
            handles[1 - bank] = start(k + 1, 1 - bank)
        for h in handles[bank]:
            h.wait()
        compute(bank)

    pltpu.sync_copy(accs, sum_out.at[wid])
    pltpu.sync_copy(accc, cnt_out.at[wid])


@jax.jit
def _sc_segsum(pred, lab):
    mesh = plsc.VectorSubcoreMesh(core_axis_name="c", subcore_axis_name="s")
    f = functools.partial(
        pl.kernel,
        out_type=[
            jax.ShapeDtypeStruct((NW, BINS), jnp.float32),
            jax.ShapeDtypeStruct((NW, BINS), jnp.float32),
        ],
        mesh=mesh,
        scratch_types=[
            pltpu.VMEM((YX, YX), jnp.float32),
            pltpu.VMEM((YX, YX), jnp.float32),
            pltpu.VMEM((YX, YX), jnp.float32),
            pltpu.VMEM((YX, YX), jnp.float32),
            pltpu.VMEM((YX, YX), jnp.float32),
            pltpu.VMEM((YX, YX), jnp.float32),
            pltpu.VMEM((BINS,), jnp.float32),
            pltpu.VMEM((BINS,), jnp.float32),
            pltpu.SemaphoreType.DMA,
            pltpu.SemaphoreType.DMA,
        ],
        compiler_params=pltpu.CompilerParams(
            needs_layout_passes=False, use_tc_tiling_on_sc=True
        ),
    )(_sc_segsum_body)
    return f(pred, lab)


def _epilogue_jnp(sums, cnts):
    S = sums.reshape(B, WPB, NLAB, LANES).sum(axis=(1, 3))
    C = cnts.reshape(B, WPB, NLAB, LANES).sum(axis=(1, 3))
    p = S / jnp.maximum(C, 1.0)
    p3 = p.reshape(B, 16, 4)[:, :, 1:]
    c3 = C.reshape(B, 16, 4)[:, :, 1:]
    gmask = (jnp.arange(16) >= 1)[None, :, None]
    pres = (c3 >= 1.0) & gmask
    neg = jnp.float32(-1e30)
    zz = jnp.where(pres, -p3 / TEMP, neg)
    m = zz.max(axis=2, keepdims=True)
    es = jnp.where(pres, jnp.exp(zz - m), 0.0)
    den = es.sum(axis=2)
    num = (p3 * es).sum(axis=2)
    score = num / jnp.maximum(den, 1e-30)
    valid = pres.sum(axis=2) >= 2
    total = jnp.where(valid, 1.0 - score, 0.0).sum()
    n = valid.sum()
    return jnp.where(n > 0, total / n, 0.0).astype(jnp.float32)


def kernel(pred, stub_label_map):
    sums, cnts = _sc_segsum(pred, stub_label_map)
    return _epilogue_jnp(sums, cnts).reshape(())


# speedup vs baseline: 1.0877x; 1.0877x over previous
import functools

import jax
import jax.numpy as jnp
from jax import lax
from jax.experimental import pallas as pl
from jax.experimental.pallas import tpu as pltpu
from jax.experimental.pallas import tpu_sc as plsc

B = 4
Z = 96
YX = 96
NW = 32
WPB = NW // B
ZPW = Z // WPB
LANES = 16
VPR = YX // LANES
NLAB = 64
BINS = NLAB * LANES
TEMP = 0.2


def _sc_segsum_body(pred_hbm, lab_hbm, sum_out, cnt_out,
                    lab0, lab1, a0, a1, b0, b1, accs, accc, sem0, sem1):
    cid = lax.axis_index("c")
    sid = lax.axis_index("s")
    wid = cid * 16 + sid
    b = wid // WPB
    z0 = (wid % WPB) * ZPW

    zeros = jnp.zeros((LANES,), jnp.float32)

    @plsc.parallel_loop(0, NLAB, unroll=8)
    def _(j):
        accs[pl.ds(j * LANES, LANES)] = zeros
        accc[pl.ds(j * LANES, LANES)] = zeros

    labs = [lab0, lab1]
    avs = [a0, a1]
    bvs = [b0, b1]
    sems = [sem0, sem1]

    def start(k, bank):
        z = z0 + k
        return (
            pltpu.async_copy(lab_hbm.at[b, 0, z], labs[bank], sems[bank]),
            pltpu.async_copy(pred_hbm.at[b, 0, z], avs[bank], sems[bank]),
            pltpu.async_copy(pred_hbm.at[b, 1, z], bvs[bank], sems[bank]),
        )

    iota = lax.iota(jnp.int32, LANES)
    ones = jnp.ones((LANES,), jnp.float32)

    def compute(bank):
        labr, ar, br = labs[bank], avs[bank], bvs[bank]

        @plsc.parallel_loop(0, YX, unroll=4)
        def _(r):
            idxs, es = [], []
            for c in range(VPR):
                sl = pl.ds(c * LANES, LANES)
                li = labr[r, sl].astype(jnp.int32)
                idxs.append(li * LANES + iota)
            for c in range(VPR):
                sl = pl.ds(c * LANES, LANES)
                d = ar[r, sl] - br[r, sl]
                es.append(jnp.exp(d))
            fgs = [1.0 / (1.0 + e) for e in es]
            for c in range(VPR):
                plsc.addupdate_scatter(accs, [idxs[c]], fgs[c])
                plsc.addupdate_scatter(accc, [idxs[c]], ones)

    handles = [None, None]
    handles[0] = start(0, 0)
    for k in range(ZPW):
        bank = k % 2
        if k + 1 < ZPW:
            handles[1 - bank] = start(k + 1, 1 - bank)
        for h in handles[bank]:
            h.wait()
        compute(bank)

    pltpu.sync_copy(accs, sum_out.at[wid])
    pltpu.sync_copy(accc, cnt_out.at[wid])


@jax.jit
def _sc_segsum(pred, lab):
    mesh = plsc.VectorSubcoreMesh(core_axis_name="c", subcore_axis_name="s")
    f = functools.partial(
        pl.kernel,
        out_type=[
            jax.ShapeDtypeStruct((NW, BINS), jnp.float32),
            jax.ShapeDtypeStruct((NW, BINS), jnp.float32),
        ],
        mesh=mesh,
        scratch_types=[
            pltpu.VMEM((YX, YX), jnp.float32),
            pltpu.VMEM((YX, YX), jnp.float32),
            pltpu.VMEM((YX, YX), jnp.float32),
            pltpu.VMEM((YX, YX), jnp.float32),
            pltpu.VMEM((YX, YX), jnp.float32),
            pltpu.VMEM((YX, YX), jnp.float32),
            pltpu.VMEM((BINS,), jnp.float32),
            pltpu.VMEM((BINS,), jnp.float32),
            pltpu.SemaphoreType.DMA,
            pltpu.SemaphoreType.DMA,
        ],
        compiler_params=pltpu.CompilerParams(
            needs_layout_passes=False, use_tc_tiling_on_sc=True
        ),
    )(_sc_segsum_body)
    return f(pred, lab)


def _epilogue_jnp(sums, cnts):
    S = sums.reshape(B, WPB, NLAB, LANES).sum(axis=(1, 3))
    C = cnts.reshape(B, WPB, NLAB, LANES).sum(axis=(1, 3))
    p = S / jnp.maximum(C, 1.0)
    p3 = p.reshape(B, 16, 4)[:, :, 1:]
    c3 = C.reshape(B, 16, 4)[:, :, 1:]
    gmask = (jnp.arange(16) >= 1)[None, :, None]
    pres = (c3 >= 1.0) & gmask
    neg = jnp.float32(-1e30)
    zz = jnp.where(pres, -p3 / TEMP, neg)
    m = zz.max(axis=2, keepdims=True)
    es = jnp.where(pres, jnp.exp(zz - m), 0.0)
    den = es.sum(axis=2)
    num = (p3 * es).sum(axis=2)
    score = num / jnp.maximum(den, 1e-30)
    valid = pres.sum(axis=2) >= 2
    total = jnp.where(valid, 1.0 - score, 0.0).sum()
    n = valid.sum()
    return jnp.where(n > 0, total / n, 0.0).astype(jnp.float32)


def kernel(pred, stub_label_map):
    sums, cnts = _sc_segsum(pred, stub_label_map)
    return _epilogue_jnp(sums, cnts).reshape(())
